# baseline (speedup 1.0000x reference)
import functools

import jax
import jax.numpy as jnp
from jax.experimental import pallas as pl
from jax.experimental.pallas import tpu as pltpu

N_NODES = 50000
N_EDGES = 800000

_C = 32
_LAYER_DIMS = [(78, 96, 64), (64, 64, 128), (128, 128, 256)]


def _eemb_body(ea_ref, wt_ref, b_ref, o_ref):
    o_ref[...] = (
        jnp.dot(ea_ref[...], wt_ref[...], preferred_element_type=jnp.float32)
        + b_ref[...]
    )


def _eemb(ea8, welt_pad, bel_pad):
    e = ea8.shape[0]
    dp = welt_pad.shape[1]
    be = 4000
    return pl.pallas_call(
        _eemb_body,
        grid=(e // be,),
        in_specs=[
            pl.BlockSpec((be, 8), lambda i: (i, 0)),
            pl.BlockSpec((8, dp), lambda i: (0, 0)),
            pl.BlockSpec((1, dp), lambda i: (0, 0)),
        ],
        out_specs=pl.BlockSpec((be, dp), lambda i: (i, 0)),
        out_shape=jax.ShapeDtypeStruct((e, dp), jnp.float32),
    )(ea8, welt_pad, bel_pad)


def _mlp_body(x_ref, a_ref, w1_ref, b1_ref, w2_ref, b2_ref, g_ref, bt_ref,
              eps_ref, o_ref, *, act):
    h0 = (1.0 + eps_ref[0, 0]) * x_ref[...] + a_ref[...]
    z = jnp.dot(h0, w1_ref[...], preferred_element_type=jnp.float32) + b1_ref[...]
    z = jnp.maximum(z, 0.0)
    h = jnp.dot(z, w2_ref[...], preferred_element_type=jnp.float32) + b2_ref[...]
    mu = jnp.mean(h, axis=-1, keepdims=True)
    var = jnp.mean((h - mu) ** 2, axis=-1, keepdims=True)
    h = (h - mu) * jax.lax.rsqrt(var + 1e-5) * g_ref[...] + bt_ref[...]
    if act:
        h = jnp.maximum(h, 0.0)
    o_ref[...] = h


def _mlp(xpad, aggr_pad, w1t_pad, b1, w2t, b2, gamma, beta, eps, act):
    n, dp = xpad.shape
    dout = w2t.shape[0]
    br = 2000
    return pl.pallas_call(
        functools.partial(_mlp_body, act=act),
        grid=(n // br,),
        in_specs=[
            pl.BlockSpec((br, dp), lambda i: (i, 0)),
            pl.BlockSpec((br, dp), lambda i: (i, 0)),
            pl.BlockSpec((dp, dout), lambda i: (0, 0)),
            pl.BlockSpec((1, dout), lambda i: (0, 0)),
            pl.BlockSpec((dout, dout), lambda i: (0, 0)),
            pl.BlockSpec((1, dout), lambda i: (0, 0)),
            pl.BlockSpec((1, dout), lambda i: (0, 0)),
            pl.BlockSpec((1, dout), lambda i: (0, 0)),
            pl.BlockSpec((1, 1), lambda i: (0, 0), memory_space=pltpu.SMEM),
        ],
        out_specs=pl.BlockSpec((br, dout), lambda i: (i, 0)),
        out_shape=jax.ShapeDtypeStruct((n, dout), jnp.float32),
    )(xpad, aggr_pad, w1t_pad, b1[None, :], w2t, b2[None, :], gamma[None, :],
      beta[None, :], eps.reshape(1, 1))


def kernel(x, edge_index, edge_attr, batch,
           l1_Wel, l1_bel, l1_W1, l1_b1, l1_W2, l1_b2, l1_gamma, l1_beta, l1_eps,
           l2_Wel, l2_bel, l2_W1, l2_b1, l2_W2, l2_b2, l2_gamma, l2_beta, l2_eps,
           l3_Wel, l3_bel, l3_W1, l3_b1, l3_W2, l3_b2, l3_gamma, l3_beta, l3_eps):
    layers = [
        (l1_Wel, l1_bel, l1_W1, l1_b1, l1_W2, l1_b2, l1_gamma, l1_beta, l1_eps, True),
        (l2_Wel, l2_bel, l2_W1, l2_b1, l2_W2, l2_b2, l2_gamma, l2_beta, l2_eps, True),
        (l3_Wel, l3_bel, l3_W1, l3_b1, l3_W2, l3_b2, l3_gamma, l3_beta, l3_eps, False),
    ]
    src = edge_index[0]
    dst = edge_index[1]
    ea8 = jnp.pad(edge_attr, ((0, 0), (0, 2)))
    h = x
    for (din, dp, dout), (wel, bel, w1, b1, w2, b2, gamma, beta, eps, act) in zip(
            _LAYER_DIMS, layers):
        welt_pad = jnp.pad(wel.T, ((0, 2), (0, dp - din)))
        bel_pad = jnp.pad(bel, (0, dp - din))[None, :]
        w1t_pad = jnp.pad(w1.T, ((0, dp - din), (0, 0)))
        xpad = jnp.pad(h, ((0, 0), (0, dp - din))) if dp != din else h
        eemb = _eemb(ea8, welt_pad, bel_pad)
        msg = jnp.maximum(xpad[src] + eemb, 0.0)
        aggr = jax.ops.segment_sum(msg, dst, num_segments=xpad.shape[0])
        h = _mlp(xpad, aggr, w1t_pad, b1, w2.T, b2, gamma, beta, eps, act)
    return (h, batch)
